# baseline (speedup 1.0000x reference)
import functools

import jax
import jax.numpy as jnp
from jax import lax
from jax.experimental import pallas as pl
from jax.experimental.pallas import tpu as pltpu
from jax.experimental.pallas import tpu_sc as plsc

_N = 100000
_D = 128
_G = 1024

_CH = 128
_NFULL = _N // _CH
_TAILR = _N - _NFULL * _CH
_NW = 32
_ROUNDS = _NFULL // _NW
_EXTRA = _NFULL - _ROUNDS * _NW
_GP = _G + 16
_BPT = _G // 16


def _count_vectors(ids_v, acc_c, nvec, pos, is15):
    def _step(j, carry):
        off = j * 16
        ids = ids_v[pl.ds(off, 16)]
        idn = ids_v[pl.ds(off + 1, 16)]
        bnd = ids != idn
        m_add = bnd | is15
        m_sub = bnd & jnp.logical_not(is15)
        plsc.addupdate_scatter(acc_c, [ids], pos, mask=m_add)
        plsc.addupdate_scatter(acc_c, [idn], -pos, mask=m_sub)
        return carry
    lax.fori_loop(0, nvec, _step, 0)


def _seg_body(x_hbm, ids_hbm, z_hbm, sums_hbm, part_c_hbm,
              xb0, xb1, xb2, id0, id1, id2, idt, idc0, idc1, idc2,
              acc_c, acc_sh, sem0, sem1, sem2):
    cid = lax.axis_index("c")
    sid = lax.axis_index("s")
    w = sid * 2 + cid

    pltpu.sync_copy(z_hbm.at[pl.ds(sid * _BPT, _BPT), :],
                    acc_sh.at[pl.ds(sid * _BPT, _BPT), :])
    z16 = jnp.zeros((16,), jnp.float32)

    def _zero(i, carry):
        acc_c[pl.ds(i * 16, 16)] = z16
        return carry
    lax.fori_loop(0, _GP // 16, _zero, 0)

    lane = lax.iota(jnp.int32, 16)
    pos = lax.convert_element_type(lane, jnp.float32) + 1.0
    is15 = lane == 15
    plsc.subcore_barrier()

    bufs = ((xb0, id0, idc0, sem0), (xb1, id1, idc1, sem1),
            (xb2, id2, idc2, sem2))

    def _fetch(rnd, slot):
        xb, idv, idc, sem = bufs[slot]
        base = (rnd * _NW + w) * _CH
        return (pltpu.async_copy(x_hbm.at[pl.ds(base, _CH), :], xb, sem),
                pltpu.async_copy(ids_hbm.at[pl.ds(base, _CH)], idv, sem),
                pltpu.async_copy(ids_hbm.at[pl.ds(base, _CH)],
                                 idc.at[pl.ds(0, _CH)], sem))

    cps = [None, None, None]
    scat = [None, None, None]
    cps[0] = _fetch(0, 0)
    cps[1] = _fetch(1, 1)
    for r in range(_ROUNDS):
        sl = r % 3
        xb, idv, idc, sem = bufs[sl]
        for c in cps[sl]:
            c.wait()
        scat[sl] = pltpu.async_copy(xb, acc_sh.at[idv], sem, add=True)
        _count_vectors(idc, acc_c, _CH // 16, pos, is15)
        if r + 2 < _ROUNDS:
            s2 = (r + 2) % 3
            if scat[s2] is not None:
                scat[s2].wait()
                scat[s2] = None
            cps[s2] = _fetch(r + 2, s2)
    for d in scat:
        if d is not None:
            d.wait()

    @pl.when(w < _EXTRA)
    def _extra():
        chunk = _ROUNDS * _NW + w
        base = chunk * _CH
        e = (pltpu.async_copy(x_hbm.at[pl.ds(base, _CH), :], xb0, sem0),
             pltpu.async_copy(ids_hbm.at[pl.ds(base, _CH)], id0, sem0),
             pltpu.async_copy(ids_hbm.at[pl.ds(base, _CH)],
                              idc0.at[pl.ds(0, _CH)], sem0))
        for c in e:
            c.wait()
        pltpu.sync_copy(xb0, acc_sh.at[id0], add=True)
        _count_vectors(idc0, acc_c, _CH // 16, pos, is15)

    @pl.when(w == _EXTRA)
    def _tail():
        base = _NFULL * _CH
        e = (pltpu.async_copy(x_hbm.at[pl.ds(base, _TAILR), :],
                              xb1.at[pl.ds(0, _TAILR), :], sem1),
             pltpu.async_copy(ids_hbm.at[pl.ds(base, _TAILR)], idt, sem1),
             pltpu.async_copy(ids_hbm.at[pl.ds(base, _TAILR)],
                              idc1.at[pl.ds(0, _TAILR)], sem1))
        for c in e:
            c.wait()
        pltpu.sync_copy(xb1.at[pl.ds(0, _TAILR), :], acc_sh.at[idt], add=True)
        _count_vectors(idc1, acc_c, _TAILR // 16, pos, is15)

    pltpu.sync_copy(acc_c, part_c_hbm.at[pl.ds(w * _GP, _GP)])
    plsc.subcore_barrier()

    pltpu.sync_copy(acc_sh.at[pl.ds(sid * _BPT, _BPT), :],
                    sums_hbm.at[pl.ds(cid * _G + sid * _BPT, _BPT), :])


def _sc_segment_sum(x, ids, zeros_rows):
    mesh = plsc.VectorSubcoreMesh(core_axis_name="c", subcore_axis_name="s")
    f = functools.partial(
        pl.kernel,
        mesh=mesh,
        compiler_params=pltpu.CompilerParams(needs_layout_passes=False),
        out_type=(
            jax.ShapeDtypeStruct((2 * _G, _D), jnp.float32),
            jax.ShapeDtypeStruct((_NW * _GP,), jnp.float32),
        ),
        scratch_types=[
            pltpu.VMEM((_CH, _D), jnp.float32),
            pltpu.VMEM((_CH, _D), jnp.float32),
            pltpu.VMEM((_CH, _D), jnp.float32),
            pltpu.VMEM((_CH,), jnp.int32),
            pltpu.VMEM((_CH,), jnp.int32),
            pltpu.VMEM((_CH,), jnp.int32),
            pltpu.VMEM((_TAILR,), jnp.int32),
            pltpu.VMEM((_CH + 16,), jnp.int32),
            pltpu.VMEM((_CH + 16,), jnp.int32),
            pltpu.VMEM((_CH + 16,), jnp.int32),
            pltpu.VMEM((_GP,), jnp.float32),
            pltpu.VMEM_SHARED((_G, _D), jnp.float32),
            pltpu.SemaphoreType.DMA,
            pltpu.SemaphoreType.DMA,
            pltpu.SemaphoreType.DMA,
        ],
    )(_seg_body)
    return f(x, ids, zeros_rows)


def _fin_body(sums_ref, cnt_ref, w_ref, b_ref, o_ref):
    s = sums_ref[pl.ds(0, _G), :] + sums_ref[pl.ds(_G, _G), :]
    row = jax.lax.dot_general(
        w_ref[...], s, (((1,), (1,)), ((), ())),
        preferred_element_type=jnp.float32)
    cnt = jnp.zeros((_G,), jnp.float32)
    for t in range(_NW):
        cnt = cnt + cnt_ref[pl.ds(t * _GP, _G)]
    pred = row[0] / jnp.maximum(cnt, 1.0) + b_ref[0, 0]
    o_ref[...] = pred.reshape(_G, 1)


def _finalize(sums2, part_c, w_row, b):
    return pl.pallas_call(
        _fin_body,
        out_shape=jax.ShapeDtypeStruct((_G, 1), jnp.float32),
    )(sums2, part_c, w_row, b.reshape(1, 1))


def kernel(x, batch, y, W, b):
    ids = batch.astype(jnp.int32)
    zeros_rows = jnp.zeros((_G, _D), jnp.float32)
    sums2, part_c = _sc_segment_sum(x, ids, zeros_rows)
    pred = _finalize(sums2, part_c, W.reshape(1, _D), b)
    return (pred, y)


# speedup vs baseline: 1.0121x; 1.0121x over previous
import functools

import jax
import jax.numpy as jnp
from jax import lax
from jax.experimental import pallas as pl
from jax.experimental.pallas import tpu as pltpu
from jax.experimental.pallas import tpu_sc as plsc

_N = 100000
_D = 128
_G = 1024

_CH = 128
_NFULL = _N // _CH
_TAILR = _N - _NFULL * _CH
_NW = 32
_ROUNDS = _NFULL // _NW
_EXTRA = _NFULL - _ROUNDS * _NW
_GP = _G + 16
_BPT = _G // 16


def _count_vectors(ids_v, acc_c, nvec, pos, is15):
    def _step(j, carry):
        off = j * 16
        ids = ids_v[pl.ds(off, 16)]
        idn = ids_v[pl.ds(off + 1, 16)]
        bnd = ids != idn
        m_add = bnd | is15
        m_sub = bnd & jnp.logical_not(is15)
        plsc.addupdate_scatter(acc_c, [ids], pos, mask=m_add)
        plsc.addupdate_scatter(acc_c, [idn], -pos, mask=m_sub)
        return carry
    lax.fori_loop(0, nvec, _step, 0)


def _seg_body(x_hbm, ids_hbm, z_hbm, sums_hbm, part_c_hbm,
              xb0, xb1, id0, id1, idt, idc0, idc1, acc_c, acc_sh,
              sem0, sem1):
    cid = lax.axis_index("c")
    sid = lax.axis_index("s")
    w = sid * 2 + cid

    pltpu.sync_copy(z_hbm.at[pl.ds(sid * _BPT, _BPT), :],
                    acc_sh.at[pl.ds(sid * _BPT, _BPT), :])
    z16 = jnp.zeros((16,), jnp.float32)

    def _zero(i, carry):
        acc_c[pl.ds(i * 16, 16)] = z16
        return carry
    lax.fori_loop(0, _GP // 16, _zero, 0)

    lane = lax.iota(jnp.int32, 16)
    pos = lax.convert_element_type(lane, jnp.float32) + 1.0
    is15 = lane == 15
    plsc.subcore_barrier()

    bufs = ((xb0, id0, idc0, sem0), (xb1, id1, idc1, sem1))

    def _fetch(chunk, slot):
        xb, idv, idc, sem = bufs[slot]
        base = chunk * _CH
        return (pltpu.async_copy(x_hbm.at[pl.ds(base, _CH), :], xb, sem),
                pltpu.async_copy(ids_hbm.at[pl.ds(base, _CH)], idv, sem),
                pltpu.async_copy(ids_hbm.at[pl.ds(base, _CH + 16)], idc, sem))

    def _consume(slot):
        xb, idv, idc, _ = bufs[slot]
        pltpu.sync_copy(xb, acc_sh.at[idv], add=True)
        _count_vectors(idc, acc_c, _CH // 16, pos, is15)

    cps = _fetch(w, 0)
    for r in range(_ROUNDS):
        nxt = None
        if r + 1 < _ROUNDS:
            nxt = _fetch((r + 1) * _NW + w, (r + 1) % 2)
        for c in cps:
            c.wait()
        _consume(r % 2)
        cps = nxt

    @pl.when(w < _EXTRA)
    def _extra():
        chunk = _ROUNDS * _NW + w
        base = chunk * _CH
        e = (pltpu.async_copy(x_hbm.at[pl.ds(base, _CH), :], xb0, sem0),
             pltpu.async_copy(ids_hbm.at[pl.ds(base, _CH)], id0, sem0),
             pltpu.async_copy(ids_hbm.at[pl.ds(base, _CH + 16)], idc0, sem0))
        for c in e:
            c.wait()
        pltpu.sync_copy(xb0, acc_sh.at[id0], add=True)
        _count_vectors(idc0, acc_c, _CH // 16, pos, is15)

    @pl.when(w == _EXTRA)
    def _tail():
        base = _NFULL * _CH
        e = (pltpu.async_copy(x_hbm.at[pl.ds(base, _TAILR), :],
                              xb1.at[pl.ds(0, _TAILR), :], sem1),
             pltpu.async_copy(ids_hbm.at[pl.ds(base, _TAILR)], idt, sem1),
             pltpu.async_copy(ids_hbm.at[pl.ds(base, _TAILR + 16)],
                              idc1.at[pl.ds(0, _TAILR + 16)], sem1))
        for c in e:
            c.wait()
        pltpu.sync_copy(xb1.at[pl.ds(0, _TAILR), :], acc_sh.at[idt], add=True)
        _count_vectors(idc1, acc_c, _TAILR // 16, pos, is15)

    pltpu.sync_copy(acc_c, part_c_hbm.at[pl.ds(w * _GP, _GP)])
    plsc.subcore_barrier()

    pltpu.sync_copy(acc_sh.at[pl.ds(sid * _BPT, _BPT), :],
                    sums_hbm.at[pl.ds(cid * _G + sid * _BPT, _BPT), :])


def _sc_segment_sum(x, ids, zeros_rows):
    mesh = plsc.VectorSubcoreMesh(core_axis_name="c", subcore_axis_name="s")
    f = functools.partial(
        pl.kernel,
        mesh=mesh,
        compiler_params=pltpu.CompilerParams(needs_layout_passes=False),
        out_type=(
            jax.ShapeDtypeStruct((2 * _G, _D), jnp.float32),
            jax.ShapeDtypeStruct((_NW * _GP,), jnp.float32),
        ),
        scratch_types=[
            pltpu.VMEM((_CH, _D), jnp.float32),
            pltpu.VMEM((_CH, _D), jnp.float32),
            pltpu.VMEM((_CH,), jnp.int32),
            pltpu.VMEM((_CH,), jnp.int32),
            pltpu.VMEM((_TAILR,), jnp.int32),
            pltpu.VMEM((_CH + 16,), jnp.int32),
            pltpu.VMEM((_CH + 16,), jnp.int32),
            pltpu.VMEM((_GP,), jnp.float32),
            pltpu.VMEM_SHARED((_G, _D), jnp.float32),
            pltpu.SemaphoreType.DMA,
            pltpu.SemaphoreType.DMA,
        ],
    )(_seg_body)
    return f(x, ids, zeros_rows)


def _fin_body(sums_ref, cnt_ref, w_ref, b_ref, o_ref):
    s = sums_ref[pl.ds(0, _G), :] + sums_ref[pl.ds(_G, _G), :]
    row = jax.lax.dot_general(
        w_ref[...], s, (((1,), (1,)), ((), ())),
        preferred_element_type=jnp.float32)
    cnt = jnp.zeros((_G,), jnp.float32)
    for t in range(_NW):
        cnt = cnt + cnt_ref[pl.ds(t * _GP, _G)]
    pred = row[0] / jnp.maximum(cnt, 1.0) + b_ref[0, 0]
    o_ref[...] = pred.reshape(_G, 1)


def _finalize(sums2, part_c, w_row, b):
    return pl.pallas_call(
        _fin_body,
        out_shape=jax.ShapeDtypeStruct((_G, 1), jnp.float32),
    )(sums2, part_c, w_row, b.reshape(1, 1))


def kernel(x, batch, y, W, b):
    ids = jnp.concatenate(
        [batch.astype(jnp.int32), jnp.full((16,), _G, jnp.int32)])
    zeros_rows = jnp.zeros((_G, _D), jnp.float32)
    sums2, part_c = _sc_segment_sum(x, ids, zeros_rows)
    pred = _finalize(sums2, part_c, W.reshape(1, _D), b)
    return (pred, y)
